# baseline (speedup 1.0000x reference)
import jax
import jax.numpy as jnp
from jax import lax
from jax.experimental import pallas as pl
from jax.experimental.pallas import tpu as pltpu
from jax.experimental.pallas import tpu_sc as plsc

NC = 2
NS = 16
LANE = 16
EPS = 1e-6


def _dot(x, w):
    return jnp.dot(x, w, preferred_element_type=jnp.float32)


def _tc_pre0_body(h_ref, eemb_ref, wa, ba, wb, bb, wd, bd, we, be, wc, bc,
                  ha_o, hb_o, hd_o, he_o, cet_o, eem_o):
    x = h_ref[...]
    ha_o[...] = _dot(x, wa[...]) + ba[...]
    hb_o[...] = _dot(x, wb[...]) + bb[...]
    hd_o[...] = _dot(x, wd[...]) + bd[...]
    he_o[...] = _dot(x, we[...]) + be[...]

    @pl.when(pl.program_id(0) == 0)
    def _():
        em = eemb_ref[...]
        ce = _dot(em, wc[...]) + bc[...]
        n = em.shape[0]
        for k in range(NC * NS):
            cet_o[pl.ds(k * n, n), :] = ce
            eem_o[pl.ds(k * n, n), :] = em


def _tc_step_body(h_ref, hA_ref, num_ref, den_ref,
                  wa, ba, wb, bb, wd, bd, we, be,
                  h_o, ha_o, hb_o, hd_o, he_o):
    h = h_ref[...] + jnp.maximum(
        hA_ref[...] + num_ref[...] / (den_ref[...] + EPS), 0.0)
    h_o[...] = h
    ha_o[...] = _dot(h, wa[...]) + ba[...]
    hb_o[...] = _dot(h, wb[...]) + bb[...]
    hd_o[...] = _dot(h, wd[...]) + bd[...]
    he_o[...] = _dot(h, we[...]) + be[...]


def _tc_fin_body(h_ref, hA_ref, num_ref, den_ref, h_o):
    h_o[...] = h_ref[...] + jnp.maximum(
        hA_ref[...] + num_ref[...] / (den_ref[...] + EPS), 0.0)


def _tc_ce_body(e_ref, wc, bc, o_ref):
    o_ref[...] = _dot(e_ref[...], wc[...]) + bc[...]


def _full(block):
    return pl.BlockSpec(block, lambda i: tuple(0 for _ in block))


def _rows(bn, w):
    return pl.BlockSpec((bn, w), lambda i: (i, 0))


def _make_sc_edge(mode_table, write_e, n_pad, E, hid, eb, split_e=False):
    per_tile = E // NS
    n_blocks = per_tile // eb
    rows_per_tile = n_pad // NS
    rc = eb
    n_chunks = rows_per_tile // rc

    mesh = plsc.VectorSubcoreMesh(core_axis_name="c", subcore_axis_name="s",
                                  num_cores=NC, num_subcores=NS)

    out_type = []
    if write_e:
        out_type.append(jax.ShapeDtypeStruct((E, hid), jnp.float32))
    out_type.append(jax.ShapeDtypeStruct((n_pad, hid), jnp.float32))
    out_type.append(jax.ShapeDtypeStruct((n_pad, hid), jnp.float32))

    scratch = [
        pltpu.VMEM((eb,), jnp.int32),
        pltpu.VMEM((eb,), jnp.int32),
        pltpu.VMEM((eb,), jnp.int32),
        pltpu.VMEM((eb, hid), jnp.float32),
        pltpu.VMEM((eb, hid), jnp.float32),
        pltpu.VMEM((eb, hid), jnp.float32),
        pltpu.VMEM((eb, hid), jnp.float32),
        pltpu.VMEM_SHARED((n_pad, hid), jnp.float32),
        pltpu.SemaphoreType.DMA,
        pltpu.SemaphoreType.DMA,
    ]
    e0n = 24 if split_e else 0
    e1n = eb - e0n
    if split_e:
        scratch.append(pltpu.VMEM((e1n, hid), jnp.float32))

    def body(*refs):
        it = iter(refs)
        src_h = next(it)
        dst_h = next(it)
        if mode_table:
            et_h = next(it)
        dh = next(it)
        eh = next(it)
        bh = next(it)
        if mode_table:
            cet = next(it)
            eintab = next(it)
        else:
            ce2 = next(it)
            if write_e:
                ein2 = next(it)
        if write_e:
            eout = next(it)
        num_out = next(it)
        den_out = next(it)
        if split_e:
            (srcb, dstb, ietb, gD, gE, aux, ceb, acc, sem, sem_e,
             einq) = it
        else:
            (srcb, dstb, ietb, gD, gE, aux, ceb, acc, sem, sem_e) = it

        c = lax.axis_index("c")
        s = lax.axis_index("s")
        is_num = c == 0

        zero16 = jnp.zeros((LANE,), jnp.float32)

        def zrow(i, carry):
            for j in range(hid // LANE):
                ceb[i, pl.ds(j * LANE, LANE)] = zero16
            return carry

        lax.fori_loop(0, rc, zrow, 0)
        for k in range(n_chunks):
            r0 = s * rows_per_tile + k * rc
            pltpu.sync_copy(ceb, acc.at[pl.ds(r0, rc)])
        plsc.subcore_barrier()

        tile_e0 = s * per_tile

        def blk(b, carry):
            base = tile_e0 + b * eb
            if write_e and split_e:
                @pl.when(is_num & (b > 0))
                def _():
                    pltpu.make_async_copy(dh.at[pl.ds(0, e0n)],
                                          einq.at[pl.ds(0, e0n)],
                                          sem_e).wait()

                @pl.when((~is_num) & (b > 0))
                def _():
                    pltpu.make_async_copy(dh.at[pl.ds(0, e1n)], einq,
                                          sem_e).wait()
            elif write_e:
                @pl.when((~is_num) & (b > 0))
                def _():
                    pltpu.make_async_copy(dh.at[pl.ds(0, eb)], aux,
                                          sem_e).wait()
            pltpu.sync_copy(src_h.at[pl.ds(base, eb)], srcb)
            pltpu.sync_copy(dst_h.at[pl.ds(base, eb)], dstb)
            if mode_table:
                pltpu.sync_copy(et_h.at[pl.ds(base, eb)], ietb)
                rep = (c * NS + s) * 16
                for k in range(eb // LANE):
                    sl = pl.ds(k * LANE, LANE)
                    ietb[sl] = ietb[sl] + rep
            cps = [pltpu.async_copy(dh.at[srcb], gD, sem),
                   pltpu.async_copy(eh.at[dstb], gE, sem)]
            if mode_table:
                cps.append(pltpu.async_copy(cet.at[ietb], ceb, sem))
            else:
                cps.append(pltpu.async_copy(ce2.at[pl.ds(base, eb)], ceb, sem))
            for cp in cps:
                cp.wait()

            @pl.when(is_num)
            def _():
                pltpu.async_copy(bh.at[srcb], aux, sem).wait()

            if write_e and split_e:
                @pl.when(is_num)
                def _():
                    pltpu.async_copy(ein2.at[pl.ds(base, e0n)],
                                     einq.at[pl.ds(0, e0n)], sem).wait()

                @pl.when(~is_num)
                def _():
                    pltpu.async_copy(ein2.at[pl.ds(base + e0n, e1n)],
                                     einq, sem).wait()

                def row_e0(i, carry2):
                    for j in range(hid // LANE):
                        sl = pl.ds(j * LANE, LANE)
                        x = gD[i, sl] + gE[i, sl] + ceb[i, sl]
                        einq[i, sl] = einq[i, sl] + jnp.maximum(x, 0.0)
                    return carry2

                def row_e1(i, carry2):
                    for j in range(hid // LANE):
                        sl = pl.ds(j * LANE, LANE)
                        x = (gD[i + e0n, sl] + gE[i + e0n, sl]
                             + ceb[i + e0n, sl])
                        einq[i, sl] = einq[i, sl] + jnp.maximum(x, 0.0)
                    return carry2

                @pl.when(is_num)
                def _():
                    lax.fori_loop(0, e0n, row_e0, 0)

                @pl.when(~is_num)
                def _():
                    lax.fori_loop(0, e1n, row_e1, 0)
            elif write_e:
                @pl.when(~is_num)
                def _():
                    if mode_table:
                        pltpu.async_copy(eintab.at[ietb], aux, sem).wait()
                    else:
                        pltpu.async_copy(ein2.at[pl.ds(base, eb)], aux,
                                         sem).wait()

            def row_num(i, carry2):
                for j in range(hid // LANE):
                    sl = pl.ds(j * LANE, LANE)
                    x = gD[i, sl] + gE[i, sl] + ceb[i, sl]
                    sg = 1.0 / (1.0 + jnp.exp(-x))
                    gD[i, sl] = sg * aux[i, sl]
                return carry2

            def row_den(i, carry2):
                for j in range(hid // LANE):
                    sl = pl.ds(j * LANE, LANE)
                    x = gD[i, sl] + gE[i, sl] + ceb[i, sl]
                    sg = 1.0 / (1.0 + jnp.exp(-x))
                    gD[i, sl] = sg
                    if write_e and not split_e:
                        aux[i, sl] = aux[i, sl] + jnp.maximum(x, 0.0)
                return carry2

            @pl.when(is_num)
            def _():
                lax.fori_loop(0, eb, row_num, 0)

            @pl.when(~is_num)
            def _():
                lax.fori_loop(0, eb, row_den, 0)

            pltpu.sync_copy(gD, acc.at[dstb], add=True)
            if write_e and split_e:
                @pl.when(is_num)
                def _():
                    pltpu.async_copy(einq.at[pl.ds(0, e0n)],
                                     eout.at[pl.ds(base, e0n)], sem_e)

                @pl.when(~is_num)
                def _():
                    pltpu.async_copy(einq,
                                     eout.at[pl.ds(base + e0n, e1n)], sem_e)
            elif write_e:
                @pl.when(~is_num)
                def _():
                    pltpu.async_copy(aux, eout.at[pl.ds(base, eb)], sem_e)
            return carry

        lax.fori_loop(0, n_blocks, blk, 0)
        if write_e and split_e:
            @pl.when(is_num)
            def _():
                pltpu.make_async_copy(dh.at[pl.ds(0, e0n)],
                                      einq.at[pl.ds(0, e0n)], sem_e).wait()

            @pl.when(~is_num)
            def _():
                pltpu.make_async_copy(dh.at[pl.ds(0, e1n)], einq,
                                      sem_e).wait()
        elif write_e:
            @pl.when(~is_num)
            def _():
                pltpu.make_async_copy(dh.at[pl.ds(0, eb)], aux, sem_e).wait()
        plsc.subcore_barrier()

        for k in range(n_chunks):
            r0 = s * rows_per_tile + k * rc
            pltpu.sync_copy(acc.at[pl.ds(r0, rc)], ceb)

            @pl.when(is_num)
            def _():
                pltpu.sync_copy(ceb, num_out.at[pl.ds(r0, rc)])

            @pl.when(~is_num)
            def _():
                pltpu.sync_copy(ceb, den_out.at[pl.ds(r0, rc)])

    return pl.kernel(body, out_type=tuple(out_type), mesh=mesh,
                     scratch_types=scratch)


def kernel(node_id, edge_index, edge_type, h_emb, e_emb,
           A_w, A_b, B_w, B_b, C_w, C_b, D_w, D_b, E_w, E_b):
    N, hid = h_emb.shape
    E = edge_index.shape[1]
    L = A_w.shape[0]
    eb = 80
    n_pad = ((N + NS * eb - 1) // (NS * eb)) * (NS * eb)
    bn = 1000
    be = 2000

    src = edge_index[0]
    dst = edge_index[1]

    w128 = _full((hid, hid))
    b128 = _full((1, hid))

    def bias(b, l):
        return b[l].reshape(1, hid)

    nsd = jax.ShapeDtypeStruct((N, hid), jnp.float32)

    n_et = e_emb.shape[0]
    rep_shape = (NC * NS * n_et, hid)
    tc_pre0 = pl.pallas_call(
        _tc_pre0_body,
        grid=(N // bn,),
        in_specs=[_rows(bn, hid), _full(e_emb.shape)] + [w128, b128] * 5,
        out_specs=[_rows(bn, hid)] * 4 + [_full(rep_shape)] * 2,
        out_shape=[nsd] * 4 + [jax.ShapeDtypeStruct(rep_shape, jnp.float32)] * 2,
    )

    tc_step = pl.pallas_call(
        _tc_step_body,
        grid=(N // bn,),
        in_specs=[_rows(bn, hid)] * 4 + [w128, b128] * 4,
        out_specs=[_rows(bn, hid)] * 5,
        out_shape=[nsd] * 5,
    )

    tc_fin = pl.pallas_call(
        _tc_fin_body,
        grid=(N // bn,),
        in_specs=[_rows(bn, hid)] * 4,
        out_specs=_rows(bn, hid),
        out_shape=nsd,
    )

    tc_ce = pl.pallas_call(
        _tc_ce_body,
        grid=(E // be,),
        in_specs=[_rows(be, hid), w128, b128],
        out_specs=_rows(be, hid),
        out_shape=jax.ShapeDtypeStruct((E, hid), jnp.float32),
    )

    sc_edge0 = _make_sc_edge(True, True, n_pad, E, hid, eb)
    sc_edge_mid = _make_sc_edge(False, True, n_pad, E, hid, eb, split_e=True)
    sc_edge_last = _make_sc_edge(False, False, n_pad, E, hid, eb)

    h = h_emb

    Ah, Bh, Dh, Eh, CeT, EemT = tc_pre0(h, e_emb,
                                        A_w[0], bias(A_b, 0),
                                        B_w[0], bias(B_b, 0),
                                        D_w[0], bias(D_b, 0),
                                        E_w[0], bias(E_b, 0),
                                        C_w[0], bias(C_b, 0))
    e_state, num, den = sc_edge0(src, dst, edge_type, Dh, Eh, Bh, CeT, EemT)

    for l in range(1, L):
        h, Ah, Bh, Dh, Eh = tc_step(h, Ah, num, den,
                                    A_w[l], bias(A_b, l), B_w[l], bias(B_b, l),
                                    D_w[l], bias(D_b, l), E_w[l], bias(E_b, l))
        Ce = tc_ce(e_state, C_w[l], bias(C_b, l))
        if l < L - 1:
            e_state, num, den = sc_edge_mid(src, dst, Dh, Eh, Bh, Ce, e_state)
        else:
            num, den = sc_edge_last(src, dst, Dh, Eh, Bh, Ce)

    return tc_fin(h, Ah, num, den)


# speedup vs baseline: 1.0466x; 1.0466x over previous
import jax
import jax.numpy as jnp
from jax import lax
from jax.experimental import pallas as pl
from jax.experimental.pallas import tpu as pltpu
from jax.experimental.pallas import tpu_sc as plsc

NC = 2
NS = 16
LANE = 16
EPS = 1e-6


def _dot(x, w):
    return jnp.dot(x, w, preferred_element_type=jnp.float32)


def _tc_pre0_body(h_ref, eemb_ref, wa, ba, wb, bb, wd, bd, we, be, wc, bc,
                  ha_o, hb_o, hd_o, he_o, cet_o, eem_o):
    x = h_ref[...]
    ha_o[...] = _dot(x, wa[...]) + ba[...]
    hb_o[...] = _dot(x, wb[...]) + bb[...]
    hd_o[...] = _dot(x, wd[...]) + bd[...]
    he_o[...] = _dot(x, we[...]) + be[...]

    @pl.when(pl.program_id(0) == 0)
    def _():
        em = eemb_ref[...]
        ce = _dot(em, wc[...]) + bc[...]
        n = em.shape[0]
        for k in range(NC * NS):
            cet_o[pl.ds(k * n, n), :] = ce
            eem_o[pl.ds(k * n, n), :] = em


def _tc_step_body(h_ref, hA_ref, num_ref, den_ref,
                  wa, ba, wb, bb, wd, bd, we, be,
                  h_o, ha_o, hb_o, hd_o, he_o):
    h = h_ref[...] + jnp.maximum(
        hA_ref[...] + num_ref[...] / (den_ref[...] + EPS), 0.0)
    h_o[...] = h
    ha_o[...] = _dot(h, wa[...]) + ba[...]
    hb_o[...] = _dot(h, wb[...]) + bb[...]
    hd_o[...] = _dot(h, wd[...]) + bd[...]
    he_o[...] = _dot(h, we[...]) + be[...]


def _tc_fin_body(h_ref, hA_ref, num_ref, den_ref, h_o):
    h_o[...] = h_ref[...] + jnp.maximum(
        hA_ref[...] + num_ref[...] / (den_ref[...] + EPS), 0.0)


def _tc_ce_body(e_ref, wc, bc, o_ref):
    o_ref[...] = _dot(e_ref[...], wc[...]) + bc[...]


def _full(block):
    return pl.BlockSpec(block, lambda i: tuple(0 for _ in block))


def _rows(bn, w):
    return pl.BlockSpec((bn, w), lambda i: (i, 0))


def _make_sc_edge(mode_table, write_e, n_pad, E, hid, eb):
    per_tile = E // NS
    n_blocks = per_tile // eb
    rows_per_tile = n_pad // NS
    rc = eb
    n_chunks = rows_per_tile // rc

    mesh = plsc.VectorSubcoreMesh(core_axis_name="c", subcore_axis_name="s",
                                  num_cores=NC, num_subcores=NS)

    out_type = []
    if write_e:
        out_type.append(jax.ShapeDtypeStruct((E, hid), jnp.float32))
    out_type.append(jax.ShapeDtypeStruct((n_pad, hid), jnp.float32))
    out_type.append(jax.ShapeDtypeStruct((n_pad, hid), jnp.float32))

    scratch = [
        pltpu.VMEM((eb,), jnp.int32),
        pltpu.VMEM((eb,), jnp.int32),
        pltpu.VMEM((eb,), jnp.int32),
        pltpu.VMEM((eb, hid), jnp.float32),
        pltpu.VMEM((eb, hid), jnp.float32),
        pltpu.VMEM((eb, hid), jnp.float32),
        pltpu.VMEM((eb, hid), jnp.float32),
        pltpu.VMEM_SHARED((n_pad, hid), jnp.float32),
        pltpu.SemaphoreType.DMA,
        pltpu.SemaphoreType.DMA,
    ]

    def body(*refs):
        it = iter(refs)
        src_h = next(it)
        dst_h = next(it)
        if mode_table:
            et_h = next(it)
        dh = next(it)
        eh = next(it)
        bh = next(it)
        if mode_table:
            cet = next(it)
            eintab = next(it)
        else:
            ce2 = next(it)
            if write_e:
                ein2 = next(it)
        if write_e:
            eout = next(it)
        num_out = next(it)
        den_out = next(it)
        (srcb, dstb, ietb, gD, gE, aux, ceb, acc, sem, sem_e) = it

        c = lax.axis_index("c")
        s = lax.axis_index("s")
        is_num = c == 0

        zero16 = jnp.zeros((LANE,), jnp.float32)

        def zrow(i, carry):
            for j in range(hid // LANE):
                ceb[i, pl.ds(j * LANE, LANE)] = zero16
            return carry

        lax.fori_loop(0, rc, zrow, 0)
        for k in range(n_chunks):
            r0 = s * rows_per_tile + k * rc
            pltpu.sync_copy(ceb, acc.at[pl.ds(r0, rc)])
        plsc.subcore_barrier()

        tile_e0 = s * per_tile

        def blk(b, carry):
            base = tile_e0 + b * eb
            if write_e:
                @pl.when((~is_num) & (b > 0))
                def _():
                    pltpu.make_async_copy(dh.at[pl.ds(0, eb)], aux,
                                          sem_e).wait()
            pltpu.sync_copy(src_h.at[pl.ds(base, eb)], srcb)
            pltpu.sync_copy(dst_h.at[pl.ds(base, eb)], dstb)
            if mode_table:
                pltpu.sync_copy(et_h.at[pl.ds(base, eb)], ietb)
                rep = (c * NS + s) * 16
                for k in range(eb // LANE):
                    sl = pl.ds(k * LANE, LANE)
                    ietb[sl] = ietb[sl] + rep
            cps = [pltpu.async_copy(dh.at[srcb], gD, sem),
                   pltpu.async_copy(eh.at[dstb], gE, sem)]
            if mode_table:
                cps.append(pltpu.async_copy(cet.at[ietb], ceb, sem))
            else:
                cps.append(pltpu.async_copy(ce2.at[pl.ds(base, eb)], ceb, sem))
            for cp in cps:
                cp.wait()

            @pl.when(is_num)
            def _():
                pltpu.async_copy(bh.at[srcb], aux, sem).wait()

            if write_e:
                @pl.when(~is_num)
                def _():
                    if mode_table:
                        pltpu.async_copy(eintab.at[ietb], aux, sem).wait()
                    else:
                        pltpu.async_copy(ein2.at[pl.ds(base, eb)], aux,
                                         sem).wait()

            def row_num(i, carry2):
                for j in range(hid // LANE):
                    sl = pl.ds(j * LANE, LANE)
                    x = gD[i, sl] + gE[i, sl] + ceb[i, sl]
                    sg = 1.0 / (1.0 + jnp.exp(-x))
                    gD[i, sl] = sg * aux[i, sl]
                return carry2

            def row_den(i, carry2):
                for j in range(hid // LANE):
                    sl = pl.ds(j * LANE, LANE)
                    x = gD[i, sl] + gE[i, sl] + ceb[i, sl]
                    sg = 1.0 / (1.0 + jnp.exp(-x))
                    gD[i, sl] = sg
                    if write_e:
                        aux[i, sl] = aux[i, sl] + jnp.maximum(x, 0.0)
                return carry2

            @pl.when(is_num)
            def _():
                lax.fori_loop(0, eb, row_num, 0)

            @pl.when(~is_num)
            def _():
                lax.fori_loop(0, eb, row_den, 0)

            pltpu.sync_copy(gD, acc.at[dstb], add=True)
            if write_e:
                @pl.when(~is_num)
                def _():
                    pltpu.async_copy(aux, eout.at[pl.ds(base, eb)], sem_e)
            return carry

        lax.fori_loop(0, n_blocks, blk, 0)
        if write_e:
            @pl.when(~is_num)
            def _():
                pltpu.make_async_copy(dh.at[pl.ds(0, eb)], aux, sem_e).wait()
        plsc.subcore_barrier()

        for k in range(n_chunks):
            r0 = s * rows_per_tile + k * rc
            pltpu.sync_copy(acc.at[pl.ds(r0, rc)], ceb)

            @pl.when(is_num)
            def _():
                pltpu.sync_copy(ceb, num_out.at[pl.ds(r0, rc)])

            @pl.when(~is_num)
            def _():
                pltpu.sync_copy(ceb, den_out.at[pl.ds(r0, rc)])

    return pl.kernel(body, out_type=tuple(out_type), mesh=mesh,
                     scratch_types=scratch)


def kernel(node_id, edge_index, edge_type, h_emb, e_emb,
           A_w, A_b, B_w, B_b, C_w, C_b, D_w, D_b, E_w, E_b):
    N, hid = h_emb.shape
    E = edge_index.shape[1]
    L = A_w.shape[0]
    eb = 80
    n_pad = ((N + NS * eb - 1) // (NS * eb)) * (NS * eb)
    bn = 1000
    be = 2000

    src = edge_index[0]
    dst = edge_index[1]

    w128 = _full((hid, hid))
    b128 = _full((1, hid))

    def bias(b, l):
        return b[l].reshape(1, hid)

    nsd = jax.ShapeDtypeStruct((N, hid), jnp.float32)

    n_et = e_emb.shape[0]
    rep_shape = (NC * NS * n_et, hid)
    tc_pre0 = pl.pallas_call(
        _tc_pre0_body,
        grid=(N // bn,),
        in_specs=[_rows(bn, hid), _full(e_emb.shape)] + [w128, b128] * 5,
        out_specs=[_rows(bn, hid)] * 4 + [_full(rep_shape)] * 2,
        out_shape=[nsd] * 4 + [jax.ShapeDtypeStruct(rep_shape, jnp.float32)] * 2,
    )

    tc_step = pl.pallas_call(
        _tc_step_body,
        grid=(N // bn,),
        in_specs=[_rows(bn, hid)] * 4 + [w128, b128] * 4,
        out_specs=[_rows(bn, hid)] * 5,
        out_shape=[nsd] * 5,
    )

    tc_fin = pl.pallas_call(
        _tc_fin_body,
        grid=(N // bn,),
        in_specs=[_rows(bn, hid)] * 4,
        out_specs=_rows(bn, hid),
        out_shape=nsd,
    )

    tc_ce = pl.pallas_call(
        _tc_ce_body,
        grid=(E // be,),
        in_specs=[_rows(be, hid), w128, b128],
        out_specs=_rows(be, hid),
        out_shape=jax.ShapeDtypeStruct((E, hid), jnp.float32),
    )

    sc_edge0 = _make_sc_edge(True, True, n_pad, E, hid, eb)
    sc_edge_mid = _make_sc_edge(False, True, n_pad, E, hid, eb)
    sc_edge_last = _make_sc_edge(False, False, n_pad, E, hid, eb)

    h = h_emb

    Ah, Bh, Dh, Eh, CeT, EemT = tc_pre0(h, e_emb,
                                        A_w[0], bias(A_b, 0),
                                        B_w[0], bias(B_b, 0),
                                        D_w[0], bias(D_b, 0),
                                        E_w[0], bias(E_b, 0),
                                        C_w[0], bias(C_b, 0))
    e_state, num, den = sc_edge0(src, dst, edge_type, Dh, Eh, Bh, CeT, EemT)

    for l in range(1, L):
        h, Ah, Bh, Dh, Eh = tc_step(h, Ah, num, den,
                                    A_w[l], bias(A_b, l), B_w[l], bias(B_b, l),
                                    D_w[l], bias(D_b, l), E_w[l], bias(E_b, l))
        Ce = tc_ce(e_state, C_w[l], bias(C_b, l))
        if l < L - 1:
            e_state, num, den = sc_edge_mid(src, dst, Dh, Eh, Bh, Ce, e_state)
        else:
            num, den = sc_edge_last(src, dst, Dh, Eh, Bh, Ce)

    return tc_fin(h, Ah, num, den)
